# baseline (speedup 1.0000x reference)
import functools

import jax
import jax.numpy as jnp
from jax import lax
from jax.experimental import pallas as pl
from jax.experimental.pallas import tpu as pltpu
from jax.experimental.pallas import tpu_sc as plsc

_B = 8192
_D = 32
_N = 8192
_R = 256
_NB = _B // _R

_NC = 2
_NS = 16
_NW = _NC * _NS
_BPW = _B // _NW


_C = 2048


def _argmin_body(x_ref, x2_ref, e_ref, e2_ref, idx_ref, acc_ref):
    mm = jnp.dot(x_ref[...], e_ref[...])
    dist = x2_ref[...] - 2.0 * mm + e2_ref[...]
    neg = -dist
    acc_v = jnp.full((_R,), -jnp.inf, jnp.float32)
    acc_f = jnp.full((_R,), -jnp.inf, jnp.float32)
    acc_i = jnp.zeros((_R,), jnp.int32)
    for c in range(_N // _C):
        chunk = neg[:, c * _C:(c + 1) * _C]
        m = jnp.max(chunk, axis=1)
        i = jnp.argmax(chunk, axis=1).astype(jnp.int32) + jnp.int32(c * _C)
        win = (m > acc_v) | ((m == acc_v) & (i < acc_i))
        acc_i = jnp.where(win, i, acc_i)
        acc_f = jnp.where(win, m, acc_f)
        acc_v = jnp.where(win, m, acc_v).astype(jnp.bfloat16).astype(jnp.float32)
    idx_ref[0, 0, :] = acc_i

    @pl.when(pl.program_id(0) == 0)
    def _():
        acc_ref[...] = jnp.zeros((1, 1), jnp.float32)

    acc_ref[...] += (-jnp.sum(acc_f)).reshape(1, 1)


_argmin_call = pl.pallas_call(
    _argmin_body,
    grid=(_NB,),
    in_specs=[
        pl.BlockSpec((_R, _D), lambda i: (i, 0)),
        pl.BlockSpec((_R, 1), lambda i: (i, 0)),
        pl.BlockSpec((_D, _N), lambda i: (0, 0)),
        pl.BlockSpec((1, _N), lambda i: (0, 0)),
    ],
    out_specs=[
        pl.BlockSpec((1, 1, _R), lambda i: (i, 0, 0)),
        pl.BlockSpec((1, 1), lambda i: (0, 0)),
    ],
    out_shape=[
        jax.ShapeDtypeStruct((_NB, 1, _R), jnp.int32),
        jax.ShapeDtypeStruct((1, 1), jnp.float32),
    ],
)

@functools.cache
def _make_sc_gather():
    mesh = plsc.VectorSubcoreMesh(core_axis_name="c", subcore_axis_name="s")

    @functools.partial(
        pl.kernel,
        mesh=mesh,
        out_type=jax.ShapeDtypeStruct((_B, _D), jnp.float32),
        scratch_types=[
            pltpu.VMEM((_BPW,), jnp.int32),
            pltpu.VMEM((_BPW, _D), jnp.float32),
            pltpu.SemaphoreType.DMA,
        ],
        compiler_params=pltpu.CompilerParams(use_tc_tiling_on_sc=False),
    )
    def _sc_gather(table_hbm, idx_hbm, out_hbm, idx_v, rows_v, sem):
        wid = lax.axis_index("s") * _NC + lax.axis_index("c")
        base = wid * _BPW
        pltpu.sync_copy(idx_hbm.at[pl.ds(base, _BPW)], idx_v)
        pltpu.async_copy(table_hbm.at[idx_v], rows_v, sem).wait()
        pltpu.sync_copy(rows_v, out_hbm.at[pl.ds(base, _BPW)])

    return _sc_gather


def kernel(input, embed):
    flatten = input.reshape(-1, _D)
    x2 = jnp.sum(flatten**2, axis=1, keepdims=True)
    e2 = jnp.sum(embed**2, axis=0, keepdims=True)
    idx3, acc = _argmin_call(flatten, x2, embed, e2)
    idx_flat = idx3.reshape(-1)
    quantize = _make_sc_gather()(embed.T, idx_flat).reshape(input.shape)
    diff = acc[0, 0] / float(_B * _D)
    embed_ind = idx_flat.reshape(input.shape[:-1])
    return (quantize, diff, embed_ind)


# speedup vs baseline: 1.1174x; 1.1174x over previous
import functools

import jax
import jax.numpy as jnp
from jax import lax
from jax.experimental import pallas as pl
from jax.experimental.pallas import tpu as pltpu
from jax.experimental.pallas import tpu_sc as plsc

_B = 8192
_D = 32
_N = 8192
_R = 512
_NB = _B // _R
_C = 2048

_NC = 2
_NS = 16
_NW = _NC * _NS
_BPW = _B // _NW


def _argmin_body(x_ref, x2_ref, e_ref, e2_ref, idx_ref, acc_ref, et_ref):
    mm = jnp.dot(x_ref[...], e_ref[...])
    dist = x2_ref[...] - 2.0 * mm + e2_ref[...]
    acc_v = jnp.full((_R,), jnp.inf, jnp.float32)
    acc_f = jnp.full((_R,), jnp.inf, jnp.float32)
    acc_i = jnp.zeros((_R,), jnp.int32)
    for c in range(_N // _C):
        chunk = dist[:, c * _C:(c + 1) * _C]
        m = jnp.min(chunk, axis=1)
        i = jnp.argmin(chunk, axis=1).astype(jnp.int32) + jnp.int32(c * _C)
        win = (m < acc_v) | ((m == acc_v) & (i < acc_i))
        acc_i = jnp.where(win, i, acc_i)
        acc_f = jnp.where(win, m, acc_f)
        acc_v = jnp.where(win, m, acc_v).astype(jnp.bfloat16).astype(jnp.float32)
    idx_ref[0, 0, :] = acc_i

    i0 = pl.program_id(0)
    et_ref[...] = e_ref[:, pl.ds(i0 * _R, _R)].T

    @pl.when(i0 == 0)
    def _():
        acc_ref[...] = jnp.zeros((1, 1), jnp.float32)

    acc_ref[...] += jnp.sum(acc_f).reshape(1, 1)


_argmin_call = pl.pallas_call(
    _argmin_body,
    grid=(_NB,),
    in_specs=[
        pl.BlockSpec((_R, _D), lambda i: (i, 0)),
        pl.BlockSpec((_R, 1), lambda i: (i, 0)),
        pl.BlockSpec((_D, _N), lambda i: (0, 0)),
        pl.BlockSpec((1, _N), lambda i: (0, 0)),
    ],
    out_specs=[
        pl.BlockSpec((1, 1, _R), lambda i: (i, 0, 0)),
        pl.BlockSpec((1, 1), lambda i: (0, 0)),
        pl.BlockSpec((_R, _D), lambda i: (i, 0)),
    ],
    out_shape=[
        jax.ShapeDtypeStruct((_NB, 1, _R), jnp.int32),
        jax.ShapeDtypeStruct((1, 1), jnp.float32),
        jax.ShapeDtypeStruct((_N, _D), jnp.float32),
    ],
)


@functools.cache
def _make_sc_gather():
    mesh = plsc.VectorSubcoreMesh(core_axis_name="c", subcore_axis_name="s")

    @functools.partial(
        pl.kernel,
        mesh=mesh,
        out_type=jax.ShapeDtypeStruct((_B, _D), jnp.float32),
        scratch_types=[
            pltpu.VMEM((_BPW,), jnp.int32),
            pltpu.VMEM((_BPW, _D), jnp.float32),
            pltpu.SemaphoreType.DMA,
        ],
        compiler_params=pltpu.CompilerParams(use_tc_tiling_on_sc=False),
    )
    def _sc_gather(table_hbm, idx_hbm, out_hbm, idx_v, rows_v, sem):
        wid = lax.axis_index("s") * _NC + lax.axis_index("c")
        base = wid * _BPW
        pltpu.sync_copy(idx_hbm.at[pl.ds(base, _BPW)], idx_v)
        pltpu.async_copy(table_hbm.at[idx_v], rows_v, sem).wait()
        pltpu.sync_copy(rows_v, out_hbm.at[pl.ds(base, _BPW)])

    return _sc_gather


def kernel(input, embed):
    flatten = input.reshape(-1, _D)
    x2 = jnp.sum(flatten**2, axis=1, keepdims=True)
    e2 = jnp.sum(embed**2, axis=0, keepdims=True)
    idx3, acc, embed_t = _argmin_call(flatten, x2, embed, e2)
    idx_flat = idx3.reshape(-1)
    quantize = _make_sc_gather()(embed_t, idx_flat).reshape(input.shape)
    diff = acc[0, 0] / float(_B * _D)
    embed_ind = idx_flat.reshape(input.shape[:-1])
    return (quantize, diff, embed_ind)


# speedup vs baseline: 1.3855x; 1.2399x over previous
import functools

import jax
import jax.numpy as jnp
from jax import lax
from jax.experimental import pallas as pl
from jax.experimental.pallas import tpu as pltpu
from jax.experimental.pallas import tpu_sc as plsc

_B = 8192
_D = 32
_N = 8192
_R = 512
_NB = _B // _R
_C = 2048

_NC = 2
_NS = 16
_NW = _NC * _NS
_BPW = _B // _NW


def _argmin_body(x_ref, e_ref, idx_ref, acc_ref, et_ref, e2_ref):
    i0 = pl.program_id(0)

    @pl.when(i0 == 0)
    def _():
        e = e_ref[...]
        e2_ref[...] = jnp.sum(e * e, axis=0, keepdims=True)
        acc_ref[...] = jnp.zeros((1, 1), jnp.float32)

    x = x_ref[...]
    x2 = jnp.sum(x * x, axis=1, keepdims=True)
    mm2 = jnp.dot(x * 2.0, e_ref[...])
    dist = x2 - mm2 + e2_ref[...]

    acc_v = jnp.full((_R,), jnp.inf, jnp.float32)
    acc_f = jnp.full((_R,), jnp.inf, jnp.float32)
    acc_c = jnp.zeros((_R,), jnp.int32)
    for c in range(_N // _C):
        m = jnp.min(dist[:, c * _C:(c + 1) * _C], axis=1)
        win = m < acc_v
        acc_c = jnp.where(win, jnp.int32(c), acc_c)
        acc_f = jnp.where(win, m, acc_f)
        acc_v = jnp.where(win, m, acc_v).astype(jnp.bfloat16).astype(jnp.float32)

    idx = jnp.full((_R,), float(_N), jnp.float32)
    for c in range(_N // _C):
        target = jnp.where(acc_c == c, acc_f, jnp.nan)
        eq = dist[:, c * _C:(c + 1) * _C] == target[:, None]
        iota = (jax.lax.broadcasted_iota(jnp.int32, (_R, _C), 1).astype(jnp.float32)
                + jnp.float32(c * _C))
        idx = jnp.minimum(idx, jnp.min(jnp.where(eq, iota, jnp.float32(_N)), axis=1))
    idx_ref[0, 0, :] = idx.astype(jnp.int32)

    et_ref[...] = e_ref[:, pl.ds(i0 * _R, _R)].T
    acc_ref[...] += jnp.sum(acc_f).reshape(1, 1)


_argmin_call = pl.pallas_call(
    _argmin_body,
    grid=(_NB,),
    in_specs=[
        pl.BlockSpec((_R, _D), lambda i: (i, 0)),
        pl.BlockSpec((_D, _N), lambda i: (0, 0)),
    ],
    out_specs=[
        pl.BlockSpec((1, 1, _R), lambda i: (i, 0, 0)),
        pl.BlockSpec((1, 1), lambda i: (0, 0)),
        pl.BlockSpec((_R, _D), lambda i: (i, 0)),
    ],
    out_shape=[
        jax.ShapeDtypeStruct((_NB, 1, _R), jnp.int32),
        jax.ShapeDtypeStruct((1, 1), jnp.float32),
        jax.ShapeDtypeStruct((_N, _D), jnp.float32),
    ],
    scratch_shapes=[pltpu.VMEM((1, _N), jnp.float32)],
)


@functools.cache
def _make_sc_gather():
    mesh = plsc.VectorSubcoreMesh(core_axis_name="c", subcore_axis_name="s")

    @functools.partial(
        pl.kernel,
        mesh=mesh,
        out_type=jax.ShapeDtypeStruct((_B, _D), jnp.float32),
        scratch_types=[
            pltpu.VMEM((_BPW,), jnp.int32),
            pltpu.VMEM((_BPW, _D), jnp.float32),
            pltpu.SemaphoreType.DMA,
        ],
        compiler_params=pltpu.CompilerParams(use_tc_tiling_on_sc=False),
    )
    def _sc_gather(table_hbm, idx_hbm, out_hbm, idx_v, rows_v, sem):
        wid = lax.axis_index("s") * _NC + lax.axis_index("c")
        base = wid * _BPW
        pltpu.sync_copy(idx_hbm.at[pl.ds(base, _BPW)], idx_v)
        pltpu.async_copy(table_hbm.at[idx_v], rows_v, sem).wait()
        pltpu.sync_copy(rows_v, out_hbm.at[pl.ds(base, _BPW)])

    return _sc_gather


def kernel(input, embed):
    flatten = input.reshape(-1, _D)
    idx3, acc, embed_t = _argmin_call(flatten, embed)
    idx_flat = idx3.reshape(-1)
    quantize = _make_sc_gather()(embed_t, idx_flat).reshape(input.shape)
    diff = acc[0, 0] / float(_B * _D)
    embed_ind = idx_flat.reshape(input.shape[:-1])
    return (quantize, diff, embed_ind)


# speedup vs baseline: 1.4013x; 1.0114x over previous
import functools

import jax
import jax.numpy as jnp
from jax import lax
from jax.experimental import pallas as pl
from jax.experimental.pallas import tpu as pltpu
from jax.experimental.pallas import tpu_sc as plsc

_B = 8192
_D = 32
_N = 8192
_R = 512
_NB = _B // _R
_C = 2048

_NC = 2
_NS = 16
_NW = _NC * _NS
_BPW = _B // _NW


def _argmin_body(x_ref, e_ref, idx_ref, acc_ref, et_ref, e2_ref):
    i0 = pl.program_id(0)

    @pl.when(i0 == 0)
    def _():
        e = e_ref[...]
        e2_ref[...] = jnp.sum(e * e, axis=0, keepdims=True)
        acc_ref[...] = jnp.zeros((1, 1), jnp.float32)

    x = x_ref[...]
    x2 = jnp.sum(x * x, axis=1, keepdims=True)
    mm2 = jnp.dot(x * 2.0, e_ref[...])
    dist = x2 - mm2 + e2_ref[...]

    acc_v = jnp.full((_R,), jnp.inf, jnp.float32)
    acc_f = jnp.full((_R,), jnp.inf, jnp.float32)
    acc_c = jnp.zeros((_R,), jnp.int32)
    for c in range(_N // _C):
        m = jnp.min(dist[:, c * _C:(c + 1) * _C], axis=1)
        win = m < acc_v
        acc_c = jnp.where(win, jnp.int32(c), acc_c)
        acc_f = jnp.where(win, m, acc_f)
        acc_v = jnp.where(win, m, acc_v).astype(jnp.bfloat16).astype(jnp.float32)

    idx = jnp.full((_R,), float(_N), jnp.float32)
    for c in range(_N // _C):
        target = jnp.where(acc_c == c, acc_f, jnp.nan)
        eq = dist[:, c * _C:(c + 1) * _C] == target[:, None]
        iota = (jax.lax.broadcasted_iota(jnp.int32, (_R, _C), 1).astype(jnp.float32)
                + jnp.float32(c * _C))
        idx = jnp.minimum(idx, jnp.min(jnp.where(eq, iota, jnp.float32(_N)), axis=1))
    idx_ref[0, 0, :] = idx.astype(jnp.int32)

    et_ref[...] = e_ref[:, pl.ds(i0 * _R, _R)].T
    acc_ref[...] += jnp.sum(acc_f).reshape(1, 1)

    @pl.when(i0 == _NB - 1)
    def _():
        acc_ref[...] *= jnp.float32(1.0 / (_B * _D))


_argmin_call = pl.pallas_call(
    _argmin_body,
    grid=(_NB,),
    in_specs=[
        pl.BlockSpec((_R, _D), lambda i: (i, 0)),
        pl.BlockSpec((_D, _N), lambda i: (0, 0)),
    ],
    out_specs=[
        pl.BlockSpec((1, 1, _R), lambda i: (i, 0, 0)),
        pl.BlockSpec((1, 1), lambda i: (0, 0)),
        pl.BlockSpec((_R, _D), lambda i: (i, 0)),
    ],
    out_shape=[
        jax.ShapeDtypeStruct((_NB, 1, _R), jnp.int32),
        jax.ShapeDtypeStruct((1, 1), jnp.float32),
        jax.ShapeDtypeStruct((_N, _D), jnp.float32),
    ],
    scratch_shapes=[pltpu.VMEM((1, _N), jnp.float32)],
)


@functools.cache
def _make_sc_gather():
    mesh = plsc.VectorSubcoreMesh(core_axis_name="c", subcore_axis_name="s")

    @functools.partial(
        pl.kernel,
        mesh=mesh,
        out_type=jax.ShapeDtypeStruct((_B, _D), jnp.float32),
        scratch_types=[
            pltpu.VMEM((_BPW,), jnp.int32),
            pltpu.VMEM((_BPW, _D), jnp.float32),
            pltpu.SemaphoreType.DMA,
        ],
        compiler_params=pltpu.CompilerParams(use_tc_tiling_on_sc=False),
    )
    def _sc_gather(table_hbm, idx_hbm, out_hbm, idx_v, rows_v, sem):
        wid = lax.axis_index("s") * _NC + lax.axis_index("c")
        base = wid * _BPW
        pltpu.sync_copy(idx_hbm.at[pl.ds(base, _BPW)], idx_v)
        pltpu.async_copy(table_hbm.at[idx_v], rows_v, sem).wait()
        pltpu.sync_copy(rows_v, out_hbm.at[pl.ds(base, _BPW)])

    return _sc_gather


def kernel(input, embed):
    flatten = input.reshape(-1, _D)
    idx3, acc, embed_t = _argmin_call(flatten, embed)
    idx_flat = idx3.reshape(-1)
    quantize = _make_sc_gather()(embed_t, idx_flat).reshape(input.shape)
    diff = acc[0, 0]
    embed_ind = idx_flat.reshape(input.shape[:-1])
    return (quantize, diff, embed_ind)


# speedup vs baseline: 1.4415x; 1.0287x over previous
import functools

import jax
import jax.numpy as jnp
from jax import lax
from jax.experimental import pallas as pl
from jax.experimental.pallas import tpu as pltpu
from jax.experimental.pallas import tpu_sc as plsc

_B = 8192
_D = 32
_N = 8192
_R = 512
_NB = _B // _R
_C = 2048

_NC = 2
_NS = 16
_NW = _NC * _NS
_BPW = _B // _NW


def _argmin_body(x_ref, e_ref, idx_ref, acc_ref, et_ref, e2_ref):
    i0 = pl.program_id(0)

    @pl.when(i0 == 0)
    def _():
        e = e_ref[...]
        e2_ref[...] = jnp.sum(e * e, axis=0, keepdims=True)
        acc_ref[...] = jnp.zeros((1, 1), jnp.float32)

    x = x_ref[...]
    x2 = jnp.sum(x * x, axis=1, keepdims=True)
    mm2 = jnp.dot(x * 2.0, e_ref[...])

    iota = jax.lax.broadcasted_iota(jnp.int32, (_R, _C), 1).astype(jnp.float32)
    acc_v = jnp.full((_R,), jnp.inf, jnp.float32)
    acc_f = jnp.full((_R,), jnp.inf, jnp.float32)
    acc_i = jnp.full((_R,), float(_N), jnp.float32)
    for c in range(_N // _C):
        sl = slice(c * _C, (c + 1) * _C)
        d_c = x2 - mm2[:, sl] + e2_ref[:, sl]
        m = jnp.min(d_c, axis=1)
        i_c = jnp.min(jnp.where(d_c == m[:, None], iota, jnp.float32(_N)), axis=1)
        i_c = i_c + jnp.float32(c * _C)
        win = m < acc_v
        acc_i = jnp.where(win, i_c, acc_i)
        acc_f = jnp.where(win, m, acc_f)
        acc_v = jnp.where(win, m, acc_v).astype(jnp.bfloat16).astype(jnp.float32)
    idx_ref[0, 0, :] = acc_i.astype(jnp.int32)

    et_ref[...] = e_ref[:, pl.ds(i0 * _R, _R)].T
    acc_ref[...] += jnp.sum(acc_f).reshape(1, 1)

    @pl.when(i0 == _NB - 1)
    def _():
        acc_ref[...] *= jnp.float32(1.0 / (_B * _D))


_argmin_call = pl.pallas_call(
    _argmin_body,
    grid=(_NB,),
    in_specs=[
        pl.BlockSpec((_R, _D), lambda i: (i, 0)),
        pl.BlockSpec((_D, _N), lambda i: (0, 0)),
    ],
    out_specs=[
        pl.BlockSpec((1, 1, _R), lambda i: (i, 0, 0)),
        pl.BlockSpec((1, 1), lambda i: (0, 0)),
        pl.BlockSpec((_R, _D), lambda i: (i, 0)),
    ],
    out_shape=[
        jax.ShapeDtypeStruct((_NB, 1, _R), jnp.int32),
        jax.ShapeDtypeStruct((1, 1), jnp.float32),
        jax.ShapeDtypeStruct((_N, _D), jnp.float32),
    ],
    scratch_shapes=[pltpu.VMEM((1, _N), jnp.float32)],
)


@functools.cache
def _make_sc_gather():
    mesh = plsc.VectorSubcoreMesh(core_axis_name="c", subcore_axis_name="s")

    @functools.partial(
        pl.kernel,
        mesh=mesh,
        out_type=jax.ShapeDtypeStruct((_B, _D), jnp.float32),
        scratch_types=[
            pltpu.VMEM((_BPW,), jnp.int32),
            pltpu.VMEM((_BPW, _D), jnp.float32),
            pltpu.SemaphoreType.DMA,
        ],
        compiler_params=pltpu.CompilerParams(use_tc_tiling_on_sc=False),
    )
    def _sc_gather(table_hbm, idx_hbm, out_hbm, idx_v, rows_v, sem):
        wid = lax.axis_index("s") * _NC + lax.axis_index("c")
        base = wid * _BPW
        pltpu.sync_copy(idx_hbm.at[pl.ds(base, _BPW)], idx_v)
        pltpu.async_copy(table_hbm.at[idx_v], rows_v, sem).wait()
        pltpu.sync_copy(rows_v, out_hbm.at[pl.ds(base, _BPW)])

    return _sc_gather


def kernel(input, embed):
    flatten = input.reshape(-1, _D)
    idx3, acc, embed_t = _argmin_call(flatten, embed)
    idx_flat = idx3.reshape(-1)
    quantize = _make_sc_gather()(embed_t, idx_flat).reshape(input.shape)
    diff = acc[0, 0]
    embed_ind = idx_flat.reshape(input.shape[:-1])
    return (quantize, diff, embed_ind)


# speedup vs baseline: 1.4497x; 1.0057x over previous
import functools

import jax
import jax.numpy as jnp
from jax import lax
from jax.experimental import pallas as pl
from jax.experimental.pallas import tpu as pltpu
from jax.experimental.pallas import tpu_sc as plsc

_B = 8192
_D = 32
_N = 8192
_R = 1024
_NB = _B // _R
_C = 2048

_NC = 2
_NS = 16
_NW = _NC * _NS
_BPW = _B // _NW


def _argmin_body(x_ref, e_ref, idx_ref, acc_ref, et_ref, e2_ref):
    i0 = pl.program_id(0)

    @pl.when(i0 == 0)
    def _():
        e = e_ref[...]
        e2_ref[...] = jnp.sum(e * e, axis=0, keepdims=True)
        acc_ref[...] = jnp.zeros((1, 1), jnp.float32)

    x = x_ref[...]
    x2 = jnp.sum(x * x, axis=1, keepdims=True)
    mm2 = jnp.dot(x * 2.0, e_ref[...])

    iota = jax.lax.broadcasted_iota(jnp.int32, (_R, _C), 1).astype(jnp.float32)
    acc_v = jnp.full((_R,), jnp.inf, jnp.float32)
    acc_f = jnp.full((_R,), jnp.inf, jnp.float32)
    acc_i = jnp.full((_R,), float(_N), jnp.float32)
    for c in range(_N // _C):
        sl = slice(c * _C, (c + 1) * _C)
        d_c = x2 - mm2[:, sl] + e2_ref[:, sl]
        m = jnp.min(d_c, axis=1)
        i_c = jnp.min(jnp.where(d_c == m[:, None], iota, jnp.float32(_N)), axis=1)
        i_c = i_c + jnp.float32(c * _C)
        win = m < acc_v
        acc_i = jnp.where(win, i_c, acc_i)
        acc_f = jnp.where(win, m, acc_f)
        acc_v = jnp.where(win, m, acc_v).astype(jnp.bfloat16).astype(jnp.float32)
    idx_ref[0, 0, :] = acc_i.astype(jnp.int32)

    et_ref[...] = e_ref[:, pl.ds(i0 * _R, _R)].T
    acc_ref[...] += jnp.sum(acc_f).reshape(1, 1)

    @pl.when(i0 == _NB - 1)
    def _():
        acc_ref[...] *= jnp.float32(1.0 / (_B * _D))


_argmin_call = pl.pallas_call(
    _argmin_body,
    grid=(_NB,),
    in_specs=[
        pl.BlockSpec((_R, _D), lambda i: (i, 0)),
        pl.BlockSpec((_D, _N), lambda i: (0, 0)),
    ],
    out_specs=[
        pl.BlockSpec((1, 1, _R), lambda i: (i, 0, 0)),
        pl.BlockSpec((1, 1), lambda i: (0, 0)),
        pl.BlockSpec((_R, _D), lambda i: (i, 0)),
    ],
    out_shape=[
        jax.ShapeDtypeStruct((_NB, 1, _R), jnp.int32),
        jax.ShapeDtypeStruct((1, 1), jnp.float32),
        jax.ShapeDtypeStruct((_N, _D), jnp.float32),
    ],
    scratch_shapes=[pltpu.VMEM((1, _N), jnp.float32)],
)


@functools.cache
def _make_sc_gather():
    mesh = plsc.VectorSubcoreMesh(core_axis_name="c", subcore_axis_name="s")

    @functools.partial(
        pl.kernel,
        mesh=mesh,
        out_type=jax.ShapeDtypeStruct((_B, _D), jnp.float32),
        scratch_types=[
            pltpu.VMEM((_BPW,), jnp.int32),
            pltpu.VMEM((_BPW, _D), jnp.float32),
            pltpu.SemaphoreType.DMA,
        ],
        compiler_params=pltpu.CompilerParams(use_tc_tiling_on_sc=False),
    )
    def _sc_gather(table_hbm, idx_hbm, out_hbm, idx_v, rows_v, sem):
        wid = lax.axis_index("s") * _NC + lax.axis_index("c")
        base = wid * _BPW
        pltpu.sync_copy(idx_hbm.at[pl.ds(base, _BPW)], idx_v)
        pltpu.async_copy(table_hbm.at[idx_v], rows_v, sem).wait()
        pltpu.sync_copy(rows_v, out_hbm.at[pl.ds(base, _BPW)])

    return _sc_gather


def kernel(input, embed):
    flatten = input.reshape(-1, _D)
    idx3, acc, embed_t = _argmin_call(flatten, embed)
    idx_flat = idx3.reshape(-1)
    quantize = _make_sc_gather()(embed_t, idx_flat).reshape(input.shape)
    diff = acc[0, 0]
    embed_ind = idx_flat.reshape(input.shape[:-1])
    return (quantize, diff, embed_ind)


# speedup vs baseline: 1.4502x; 1.0004x over previous
import functools

import jax
import jax.numpy as jnp
from jax import lax
from jax.experimental import pallas as pl
from jax.experimental.pallas import tpu as pltpu
from jax.experimental.pallas import tpu_sc as plsc

_B = 8192
_D = 32
_N = 8192
_R = 1024
_NB = _B // _R
_C = 2048

_NC = 2
_NS = 16
_NW = _NC * _NS
_BPW = _B // _NW


def _argmin_body(x_ref, e_ref, idx_ref, acc_ref, et_ref, e2_ref):
    i0 = pl.program_id(0)

    @pl.when(i0 == 0)
    def _():
        e = e_ref[...]
        e2_ref[...] = jnp.sum(e * e, axis=0, keepdims=True)
        acc_ref[...] = jnp.zeros((1, 1), jnp.float32)

    x = x_ref[...]
    x2 = jnp.sum(x * x, axis=1, keepdims=True)
    x_2 = x * 2.0

    iota = jax.lax.broadcasted_iota(jnp.int32, (_R, _C), 1).astype(jnp.float32)
    acc_v = jnp.full((_R,), jnp.inf, jnp.float32)
    acc_f = jnp.full((_R,), jnp.inf, jnp.float32)
    acc_i = jnp.full((_R,), float(_N), jnp.float32)
    for c in range(_N // _C):
        sl = slice(c * _C, (c + 1) * _C)
        mm2_c = jnp.dot(x_2, e_ref[:, sl])
        d_c = x2 - mm2_c + e2_ref[:, sl]
        m = jnp.min(d_c, axis=1)
        i_c = jnp.min(jnp.where(d_c == m[:, None], iota, jnp.float32(_N)), axis=1)
        i_c = i_c + jnp.float32(c * _C)
        win = m < acc_v
        acc_i = jnp.where(win, i_c, acc_i)
        acc_f = jnp.where(win, m, acc_f)
        acc_v = jnp.where(win, m, acc_v).astype(jnp.bfloat16).astype(jnp.float32)
    idx_ref[0, 0, :] = acc_i.astype(jnp.int32)

    et_ref[...] = e_ref[:, pl.ds(i0 * _R, _R)].T
    acc_ref[...] += jnp.sum(acc_f).reshape(1, 1)

    @pl.when(i0 == _NB - 1)
    def _():
        acc_ref[...] *= jnp.float32(1.0 / (_B * _D))


_argmin_call = pl.pallas_call(
    _argmin_body,
    grid=(_NB,),
    in_specs=[
        pl.BlockSpec((_R, _D), lambda i: (i, 0)),
        pl.BlockSpec((_D, _N), lambda i: (0, 0)),
    ],
    out_specs=[
        pl.BlockSpec((1, 1, _R), lambda i: (i, 0, 0)),
        pl.BlockSpec((1, 1), lambda i: (0, 0)),
        pl.BlockSpec((_R, _D), lambda i: (i, 0)),
    ],
    out_shape=[
        jax.ShapeDtypeStruct((_NB, 1, _R), jnp.int32),
        jax.ShapeDtypeStruct((1, 1), jnp.float32),
        jax.ShapeDtypeStruct((_N, _D), jnp.float32),
    ],
    scratch_shapes=[pltpu.VMEM((1, _N), jnp.float32)],
)


@functools.cache
def _make_sc_gather():
    mesh = plsc.VectorSubcoreMesh(core_axis_name="c", subcore_axis_name="s")

    @functools.partial(
        pl.kernel,
        mesh=mesh,
        out_type=jax.ShapeDtypeStruct((_B, _D), jnp.float32),
        scratch_types=[
            pltpu.VMEM((_BPW,), jnp.int32),
            pltpu.VMEM((_BPW, _D), jnp.float32),
            pltpu.SemaphoreType.DMA,
        ],
        compiler_params=pltpu.CompilerParams(use_tc_tiling_on_sc=False),
    )
    def _sc_gather(table_hbm, idx_hbm, out_hbm, idx_v, rows_v, sem):
        wid = lax.axis_index("s") * _NC + lax.axis_index("c")
        base = wid * _BPW
        pltpu.sync_copy(idx_hbm.at[pl.ds(base, _BPW)], idx_v)
        pltpu.async_copy(table_hbm.at[idx_v], rows_v, sem).wait()
        pltpu.sync_copy(rows_v, out_hbm.at[pl.ds(base, _BPW)])

    return _sc_gather


def kernel(input, embed):
    flatten = input.reshape(-1, _D)
    idx3, acc, embed_t = _argmin_call(flatten, embed)
    idx_flat = idx3.reshape(-1)
    quantize = _make_sc_gather()(embed_t, idx_flat).reshape(input.shape)
    diff = acc[0, 0]
    embed_ind = idx_flat.reshape(input.shape[:-1])
    return (quantize, diff, embed_ind)
